# baseline (speedup 1.0000x reference)
import functools

import jax
import jax.numpy as jnp
from jax import lax
from jax.experimental import pallas as pl
from jax.experimental.pallas import tpu as pltpu
from jax.experimental.pallas import tpu_sc as plsc

_LANES = 16
_IDX_PIECE = 128


@functools.lru_cache(maxsize=None)
def _build(B, C, V, D):
    info = plsc.get_sparse_core_info()
    NC, NS = info.num_cores, info.num_subcores
    NW = NC * NS
    R = B // NW
    CR = 128
    NCH = R // CR
    NIDX = CR * C
    NVEC = NIDX // _LANES
    NPIECE = NIDX // _IDX_PIECE
    assert B % NW == 0 and R % CR == 0 and NIDX % _IDX_PIECE == 0

    mesh = plsc.VectorSubcoreMesh(core_axis_name="c", subcore_axis_name="s")

    @functools.partial(
        pl.kernel,
        mesh=mesh,
        out_type=jax.ShapeDtypeStruct((B * C, D), jnp.float32),
        compiler_params=pltpu.CompilerParams(use_tc_tiling_on_sc=False),
        scratch_types=[
            pltpu.VMEM((NIDX,), jnp.int32),
            pltpu.VMEM((NIDX,), jnp.int32),
            pltpu.VMEM((NIDX, D), jnp.float32),
            pltpu.SemaphoreType.DMA,
        ],
    )
    def gather_kernel(x_hbm, tab_hbm, out_hbm, xv, fv, rows, sem):
        wid = lax.axis_index("s") * NC + lax.axis_index("c")
        lane = lax.iota(jnp.int32, _LANES)

        def chunk(j, carry):
            p0 = pl.multiple_of((wid * NCH + j) * NIDX, 8)
            pltpu.sync_copy(x_hbm.at[pl.ds(p0, NIDX)], xv)
            for t in range(NVEC):
                col = (lane + (t * _LANES)) % C
                fv[pl.ds(t * _LANES, _LANES)] = (
                    xv[pl.ds(t * _LANES, _LANES)] + col * V
                )
            copies = [
                pltpu.async_copy(
                    tab_hbm.at[fv.at[pl.ds(p * _IDX_PIECE, _IDX_PIECE)]],
                    rows.at[pl.ds(p * _IDX_PIECE, _IDX_PIECE)],
                    sem,
                )
                for p in range(NPIECE)
            ]
            for cp in copies:
                cp.wait()
            pltpu.sync_copy(rows, out_hbm.at[pl.ds(p0, NIDX)])
            return carry

        lax.fori_loop(0, NCH, chunk, 0)

    return gather_kernel


def kernel(x_batch, tables):
    B, C = x_batch.shape
    _, V, D = tables.shape
    x_flat = x_batch.reshape(B * C)
    tab = tables.reshape(C * V, D)
    out = _build(B, C, V, D)(x_flat, tab)
    return out.reshape(B, C * D)


# speedup vs baseline: 1.0005x; 1.0005x over previous
import functools

import jax
import jax.numpy as jnp
from jax import lax
from jax.experimental import pallas as pl
from jax.experimental.pallas import tpu as pltpu
from jax.experimental.pallas import tpu_sc as plsc

_LANES = 16
_IDX_PIECE = 128


@functools.lru_cache(maxsize=None)
def _build(B, C, V, D):
    info = plsc.get_sparse_core_info()
    NC, NS = info.num_cores, info.num_subcores
    NW = NC * NS
    R = B // NW
    CR = 128
    NCH = R // CR
    NIDX = CR * C
    NVEC = NIDX // _LANES
    NPIECE = NIDX // _IDX_PIECE
    assert B % NW == 0 and R % CR == 0 and NIDX % _IDX_PIECE == 0

    mesh = plsc.VectorSubcoreMesh(core_axis_name="c", subcore_axis_name="s")

    @functools.partial(
        pl.kernel,
        mesh=mesh,
        out_type=jax.ShapeDtypeStruct((B * C, D), jnp.float32),
        compiler_params=pltpu.CompilerParams(use_tc_tiling_on_sc=False),
        scratch_types=[
            pltpu.VMEM((NIDX,), jnp.int32),
            pltpu.VMEM((NIDX,), jnp.int32),
            pltpu.VMEM((NIDX, D), jnp.float32),
            pltpu.SemaphoreType.DMA,
        ],
    )
    def gather_kernel(x_hbm, tab_hbm, out_hbm, xv, fv, rows, sem):
        wid = lax.axis_index("s") * NC + lax.axis_index("c")
        lane = lax.iota(jnp.int32, _LANES)

        def chunk(j, carry):
            p0 = pl.multiple_of((wid * NCH + j) * NIDX, 8)
            pltpu.sync_copy(x_hbm.at[pl.ds(p0, NIDX)], xv)
            for t in range(NVEC):
                col = (lane + (t * _LANES)) % C
                fv[pl.ds(t * _LANES, _LANES)] = (
                    xv[pl.ds(t * _LANES, _LANES)] + col * V
                )
            pltpu.async_copy(tab_hbm.at[fv], rows, sem).wait()
            pltpu.sync_copy(rows, out_hbm.at[pl.ds(p0, NIDX)])
            return carry

        lax.fori_loop(0, NCH, chunk, 0)

    return gather_kernel


def kernel(x_batch, tables):
    B, C = x_batch.shape
    _, V, D = tables.shape
    x_flat = x_batch.reshape(B * C)
    tab = tables.reshape(C * V, D)
    out = _build(B, C, V, D)(x_flat, tab)
    return out.reshape(B, C * D)
